# baseline (speedup 1.0000x reference)
import functools

import jax
import jax.numpy as jnp
from jax import lax
from jax.experimental import pallas as pl
from jax.experimental.pallas import tpu as pltpu
from jax.experimental.pallas import tpu_sc as plsc

_NUM_CORES = 2
_NUM_SUBCORES = 16
_NUM_WORKERS = _NUM_CORES * _NUM_SUBCORES
_CHUNK_ROWS = 32
_CHUNKS_PER_WORKER = 4


def _sc_copy(emb_hbm, out_hbm, buf0, buf1, sem0, sem1):
    wid = lax.axis_index("s") * _NUM_CORES + lax.axis_index("c")
    base = wid * (_CHUNK_ROWS * _CHUNKS_PER_WORKER)
    bufs = (buf0, buf1)
    sems = (sem0, sem1)
    pltpu.make_async_copy(
        emb_hbm.at[pl.ds(base, _CHUNK_ROWS), :], bufs[0], sems[0]
    ).start()
    for i in range(_CHUNKS_PER_WORKER):
        row = base + i * _CHUNK_ROWS
        if i + 1 < _CHUNKS_PER_WORKER:
            nxt = base + (i + 1) * _CHUNK_ROWS
            pltpu.make_async_copy(
                emb_hbm.at[pl.ds(nxt, _CHUNK_ROWS), :],
                bufs[(i + 1) % 2],
                sems[(i + 1) % 2],
            ).start()
        pltpu.make_async_copy(
            emb_hbm.at[pl.ds(row, _CHUNK_ROWS), :], bufs[i % 2], sems[i % 2]
        ).wait()
        pltpu.sync_copy(bufs[i % 2], out_hbm.at[pl.ds(row, _CHUNK_ROWS), :])


def kernel(x, embedding):
    T = x.shape[1]
    F = embedding.shape[1]
    mesh = plsc.VectorSubcoreMesh(core_axis_name="c", subcore_axis_name="s")
    sc_copy = functools.partial(
        pl.kernel,
        mesh=mesh,
        out_type=jax.ShapeDtypeStruct((T, F), embedding.dtype),
        scratch_types=[
            pltpu.VMEM((_CHUNK_ROWS, F), embedding.dtype),
            pltpu.VMEM((_CHUNK_ROWS, F), embedding.dtype),
            pltpu.SemaphoreType.DMA,
            pltpu.SemaphoreType.DMA,
        ],
    )(_sc_copy)
    return sc_copy(embedding)


# speedup vs baseline: 1.0069x; 1.0069x over previous
import functools

import jax
import jax.numpy as jnp
from jax import lax
from jax.experimental import pallas as pl
from jax.experimental.pallas import tpu as pltpu
from jax.experimental.pallas import tpu_sc as plsc

_NUM_CORES = 2
_NUM_SUBCORES = 16
_NUM_WORKERS = _NUM_CORES * _NUM_SUBCORES
_CHUNK_ROWS = 32
_CHUNKS_PER_WORKER = 4
_NBUF = 3


def _read_copy(emb_hbm, buf, row):
    return pltpu.make_async_copy(
        emb_hbm.at[pl.ds(row, _CHUNK_ROWS), :], buf[0], buf[1]
    )


def _write_copy(out_hbm, buf, row):
    return pltpu.make_async_copy(
        buf[0], out_hbm.at[pl.ds(row, _CHUNK_ROWS), :], buf[2]
    )


def _sc_copy(emb_hbm, out_hbm, b0, b1, b2, r0, r1, r2, w0, w1, w2):
    wid = lax.axis_index("s") * _NUM_CORES + lax.axis_index("c")
    base = wid * (_CHUNK_ROWS * _CHUNKS_PER_WORKER)
    bufs = ((b0, r0, w0), (b1, r1, w1), (b2, r2, w2))
    for i in range(_NBUF):
        _read_copy(emb_hbm, bufs[i], base + i * _CHUNK_ROWS).start()
    for i in range(_CHUNKS_PER_WORKER):
        buf = bufs[i % _NBUF]
        row = base + i * _CHUNK_ROWS
        if i >= _NBUF:
            _write_copy(out_hbm, buf, base + (i - _NBUF) * _CHUNK_ROWS).wait()
            _read_copy(emb_hbm, buf, row).start()
        _read_copy(emb_hbm, buf, row).wait()
        _write_copy(out_hbm, buf, row).start()
    for i in range(max(0, _CHUNKS_PER_WORKER - _NBUF), _CHUNKS_PER_WORKER):
        buf = bufs[i % _NBUF]
        _write_copy(out_hbm, buf, base + i * _CHUNK_ROWS).wait()


def kernel(x, embedding):
    T = x.shape[1]
    F = embedding.shape[1]
    mesh = plsc.VectorSubcoreMesh(core_axis_name="c", subcore_axis_name="s")
    sc_copy = functools.partial(
        pl.kernel,
        mesh=mesh,
        out_type=jax.ShapeDtypeStruct((T, F), embedding.dtype),
        scratch_types=(
            [pltpu.VMEM((_CHUNK_ROWS, F), embedding.dtype)] * _NBUF
            + [pltpu.SemaphoreType.DMA] * (2 * _NBUF)
        ),
    )(_sc_copy)
    return sc_copy(embedding)
